# Initial kernel scaffold; baseline (speedup 1.0000x reference)
import jax
import jax.numpy as jnp
from jax.experimental import pallas as pl


def kernel(node_attr, node_t_emb, edge_index, edge_attr_, edge_extra, edge_sh, score_norm, params):
    raise NotImplementedError("write your pallas kernel here")


# speedup vs baseline: 85.1879x; 85.1879x over previous
import functools

import jax
import jax.numpy as jnp
from jax import lax
from jax.experimental import pallas as pl
from jax.experimental.pallas import tpu as pltpu
from jax.experimental.pallas import tpu_sc as plsc

NS = 16
SUB = 80


def _sc_gather(tbl_s, tbl_d, src, dst):
    n, cs = tbl_s.shape
    cd = tbl_d.shape[1]
    e = src.shape[0]
    nw = 32
    per_w = e // nw
    chk = 2000
    nchunks = per_w // chk
    mesh = plsc.VectorSubcoreMesh(core_axis_name="c", subcore_axis_name="s")

    @functools.partial(
        pl.kernel,
        out_type=(jax.ShapeDtypeStruct((e, cs), jnp.float32),
                  jax.ShapeDtypeStruct((e, cd), jnp.float32)),
        mesh=mesh,
        compiler_params=pltpu.CompilerParams(use_tc_tiling_on_sc=False),
        scratch_types=(pltpu.VMEM((chk,), jnp.int32),
                       pltpu.VMEM((chk,), jnp.int32),
                       pltpu.VMEM((chk, cs), jnp.float32),
                       pltpu.VMEM((chk, cd), jnp.float32),
                       pltpu.SemaphoreType.DMA,
                       pltpu.SemaphoreType.DMA),
    )
    def gk(tbl_s_hbm, tbl_d_hbm, src_hbm, dst_hbm, outs_hbm, outd_hbm,
           idxs_v, idxd_v, rs_v, rd_v, sem_s, sem_d):
        wid = lax.axis_index("s") * 2 + lax.axis_index("c")
        base = wid * per_w

        def chunk(i, carry):
            off = base + i * chk
            pltpu.sync_copy(src_hbm.at[pl.ds(off, chk)], idxs_v)
            pltpu.sync_copy(dst_hbm.at[pl.ds(off, chk)], idxd_v)
            a = pltpu.async_copy(tbl_s_hbm.at[idxs_v], rs_v, sem_s)
            b = pltpu.async_copy(tbl_d_hbm.at[idxd_v], rd_v, sem_d)
            a.wait()
            b.wait()
            pltpu.sync_copy(rs_v, outs_hbm.at[pl.ds(off, chk)])
            pltpu.sync_copy(rd_v, outd_hbm.at[pl.ds(off, chk)])
            return carry

        lax.fori_loop(0, nchunks, chunk, 0)

    return gk(tbl_s, tbl_d, src, dst)


def _sc_scatter_add(m, dst, zeros, n):
    _, e, c = m.shape
    per_t = e // 16
    chk = 2000
    nchunks = per_t // chk
    nrt = n // 16
    mesh = plsc.VectorSubcoreMesh(core_axis_name="c", subcore_axis_name="s")

    @functools.partial(
        pl.kernel,
        out_type=jax.ShapeDtypeStruct((2, n, c), jnp.float32),
        mesh=mesh,
        compiler_params=pltpu.CompilerParams(use_tc_tiling_on_sc=False),
        scratch_types=(pltpu.VMEM_SHARED((n, c), jnp.float32),
                       pltpu.VMEM((chk,), jnp.int32),
                       pltpu.VMEM((chk, c), jnp.float32),
                       pltpu.SemaphoreType.DMA),
    )
    def sk(m_hbm, dst_hbm, z_hbm, out_hbm, agg_sh, idx_v, rows_v, sem):
        ci = lax.axis_index("c")
        s = lax.axis_index("s")
        pltpu.sync_copy(z_hbm.at[pl.ds(s * nrt, nrt)],
                        agg_sh.at[pl.ds(s * nrt, nrt)])
        plsc.subcore_barrier()

        def chunk(i, carry):
            off = s * per_t + i * chk
            pltpu.sync_copy(dst_hbm.at[pl.ds(off, chk)], idx_v)
            pltpu.sync_copy(m_hbm.at[ci, pl.ds(off, chk)], rows_v)
            pltpu.async_copy(rows_v, agg_sh.at[idx_v], sem, add=True).wait()
            return carry

        lax.fori_loop(0, nchunks, chunk, 0)
        plsc.subcore_barrier()
        pltpu.sync_copy(agg_sh.at[pl.ds(s * nrt, nrt)],
                        out_hbm.at[ci, pl.ds(s * nrt, nrt)])

    return sk(m, dst, zeros)


def _ln(v, g, b):
    mu = jnp.mean(v, axis=-1, keepdims=True)
    d = v - mu
    var = jnp.mean(d * d, axis=-1, keepdims=True)
    return d * lax.rsqrt(var + 1e-5) * g + b


def _node0_body(na_ref, te_ref, lng_ref, lnb_ref, w1t_ref, w1a_ref, b1_ref,
                w2_ref, b2_ref, w3_ref, b3_ref, k0_ref, tbl_ref, xd_ref):
    xl = _ln(na_ref[...], lng_ref[...], lnb_ref[...])
    h = jnp.maximum(te_ref[...] @ w1t_ref[...] + xl @ w1a_ref[...] + b1_ref[...], 0.0)
    h = jnp.maximum(h @ w2_ref[...] + b2_ref[...], 0.0)
    x0 = h @ w3_ref[...] + b3_ref[...]
    xd_ref[...] = x0
    tbl_ref[...] = x0 @ k0_ref[...]


def _edge0_body(eat_ref, eex_ref, sh_ref, gs_ref, gd_ref,
                lng_ref, lnb_ref, ew1x_ref, ew1e_ref, eb1_ref, ew2_ref,
                eb2_ref, ew3_ref, eb3_ref,
                fa_ref, fb_ref, fc_ref, fb1_ref, fw2_ref, fb2_ref, fw3_ref,
                fb3_ref, r_ref, t_ref, pa_ref, ba_ref, pb_ref,
                e_ref, m_ref):
    el = _ln(eat_ref[...], lng_ref[...], lnb_ref[...])
    h = jnp.maximum(eex_ref[...] @ ew1x_ref[...] + el @ ew1e_ref[...] + eb1_ref[...], 0.0)
    h = jnp.maximum(h @ ew2_ref[...] + eb2_ref[...], 0.0)
    ee = h @ ew3_ref[...] + eb3_ref[...]
    e_ref[...] = ee
    gs = gs_ref[...]
    h2 = jnp.maximum(ee @ fa_ref[...] + gs[:, :NS] @ fb_ref[...]
                     + gd_ref[...] @ fc_ref[...] + fb1_ref[...], 0.0)
    h2 = jnp.maximum(h2 @ fw2_ref[...] + fb2_ref[...], 0.0)
    wf = h2 @ fw3_ref[...] + fb3_ref[...]
    mm = (gs[:, NS:] @ r_ref[...]) * (sh_ref[...] @ t_ref[...]) * wf
    m_ref[0] = mm @ pa_ref[...] + ba_ref[...]
    m_ref[1] = mm @ pb_ref[...]


def _edge1_body(ee_ref, sh_ref, gs_ref, gd_ref,
                fa_ref, fb_ref, fc_ref, fb1_ref, fw2_ref, fb2_ref, fw3_ref,
                fb3_ref, r_ref, t_ref, pa_ref, pb_ref, m_ref):
    gs = gs_ref[...]
    h2 = jnp.maximum(ee_ref[...] @ fa_ref[...] + gs[:, :NS] @ fb_ref[...]
                     + gd_ref[...] @ fc_ref[...] + fb1_ref[...], 0.0)
    h2 = jnp.maximum(h2 @ fw2_ref[...] + fb2_ref[...], 0.0)
    wf = h2 @ fw3_ref[...] + fb3_ref[...]
    mm = (gs[:, NS:NS + 14] @ r_ref[...]) * (sh_ref[...] @ t_ref[...]) * wf
    m_ref[0] = mm @ pa_ref[...]
    m_ref[1] = mm @ pb_ref[...]


def _node1_body(agg_ref, xd0_ref, loa_ref, lob_ref, pad_ref, g1_ref, sel_ref,
                x1_ref, tbl_ref, xd1_ref, deg_ref):
    agga = agg_ref[0]
    aggb = agg_ref[1]
    deg = jnp.maximum(agga[:, 7:8], 1.0)
    out = (agga[:, :7] @ loa_ref[...] + aggb[:, :7] @ lob_ref[...]) / deg
    x1 = out + xd0_ref[...] @ pad_ref[...]
    x1_ref[...] = x1
    tbl_ref[...] = x1 @ g1_ref[...]
    xd1_ref[...] = x1 @ sel_ref[...]
    deg_ref[...] = deg


def _node2_body(agg_ref, x1_ref, deg_ref, sn_ref, loa_ref, lob_ref, pad_ref,
                wt_ref, o_ref):
    agga = agg_ref[0]
    aggb = agg_ref[1]
    out = (agga[:, :10] @ loa_ref[...] + aggb[:, :10] @ lob_ref[...]) / deg_ref[...]
    x2 = out + x1_ref[...] @ pad_ref[...]
    t = jnp.dot(x2, wt_ref[...], preferred_element_type=jnp.float32)
    cols = [jnp.sum(x2 * t[:, k * 40:(k + 1) * 40], axis=-1, keepdims=True)
            for k in range(6)]
    o6 = jnp.concatenate(cols, axis=-1)
    o_ref[...] = (o6[:, :3] + o6[:, 3:]) * 0.5 * sn_ref[...]


def _bspec(block, ndim):
    return pl.BlockSpec(block, lambda i, _b=ndim: (i,) + (0,) * (_b - 1))


def _wspec(arr):
    return pl.BlockSpec(arr.shape, lambda i, _n=arr.ndim: (0,) * _n)


def _tc_call(body, grid, blocked, weights, out_blocks, out_shapes):
    in_specs = [s for _, s in blocked] + [_wspec(w) for w in weights]
    return pl.pallas_call(
        body,
        grid=grid,
        in_specs=in_specs,
        out_specs=out_blocks,
        out_shape=out_shapes,
    )(*[a for a, _ in blocked], *weights)


def kernel(node_attr, node_t_emb, edge_index, edge_attr_, edge_extra,
           edge_sh, score_norm, params):
    f32 = jnp.float32
    n = node_attr.shape[0]
    e = edge_sh.shape[0]
    p = params
    src1 = edge_index[0]
    dst1 = edge_index[1]

    bn = 2000
    be = 4000
    gn = n // bn
    ge = e // be

    nW1, nb1, nW2, nb2, nW3, nb3 = p['node_mlp']
    eW1, ebb1, eW2, ebb2, eW3, ebb3 = p['edge_mlp']
    lin_in0 = p['lin_in_0']
    lin_in1 = p['lin_in_1']
    f0W1, f0b1, f0W2, f0b2, f0W3, f0b3 = p['fc_0']
    f1W1, f1b1, f1W2, f1b2, f1W3, f1b3 = p['fc_1']
    P0 = p['P_0']
    P1 = p['P_1']
    lo0 = p['lin_out_0']
    lo1 = p['lin_out_1']

    row = lambda v: v.reshape(1, -1).astype(f32)
    K0 = jnp.concatenate([jnp.eye(16, dtype=f32), lin_in0], axis=1)
    R0 = jnp.kron(jnp.eye(8, dtype=f32), jnp.ones((1, 4), f32))
    T0 = jnp.tile(jnp.eye(4, dtype=f32), (1, 8))
    Pa0 = jnp.concatenate([P0[:, :7], jnp.zeros((32, 1), f32)], axis=1)
    ba0 = jnp.concatenate([jnp.zeros((1, 7), f32), jnp.ones((1, 1), f32)], axis=1)
    Pb0 = jnp.concatenate([P0[:, 7:], jnp.zeros((32, 1), f32)], axis=1)
    R1 = jnp.kron(jnp.eye(14, dtype=f32), jnp.ones((1, 4), f32))
    T1 = jnp.tile(jnp.eye(4, dtype=f32), (1, 14))
    Pa1 = jnp.concatenate([P1[:, :10], jnp.zeros((56, 2), f32)], axis=1)
    Pb1 = jnp.concatenate([P1[:, 10:], jnp.zeros((56, 2), f32)], axis=1)
    G1 = jnp.concatenate([jnp.eye(28, 16, dtype=f32), lin_in1,
                          jnp.zeros((28, 2), f32)], axis=1)
    SEL1 = jnp.eye(28, 16, dtype=f32)
    PAD0 = jnp.eye(16, 28, dtype=f32)
    PAD1 = jnp.eye(28, 40, dtype=f32)
    Wt2 = p['Wtp'].transpose(1, 2, 0).reshape(40, 240)

    tbl0, xd0 = _tc_call(
        _node0_body, (gn,),
        [(node_attr, _bspec((bn, 32), 2)), (node_t_emb, _bspec((bn, 32), 2))],
        [row(p['ln_node_g']), row(p['ln_node_b']),
         nW1[:32], nW1[32:], row(nb1), nW2, row(nb2), nW3, row(nb3), K0],
        [_bspec((bn, 24), 2), _bspec((bn, 16), 2)],
        [jax.ShapeDtypeStruct((n, 24), f32), jax.ShapeDtypeStruct((n, 16), f32)],
    )

    gs0, gd0 = _sc_gather(tbl0, xd0, src1, dst1)
    eemb, m0 = _tc_call(
        _edge0_body, (ge,),
        [(edge_attr_, _bspec((be, 32), 2)), (edge_extra, _bspec((be, 80), 2)),
         (edge_sh, _bspec((be, 4), 2)), (gs0, _bspec((be, 24), 2)),
         (gd0, _bspec((be, 16), 2))],
        [row(p['ln_edge_g']), row(p['ln_edge_b']),
         eW1[:80], eW1[80:], row(ebb1), eW2, row(ebb2), eW3, row(ebb3),
         f0W1[:16], f0W1[16:32], f0W1[32:], row(f0b1), f0W2, row(f0b2),
         f0W3, row(f0b3), R0, T0, Pa0, ba0, Pb0],
        [_bspec((be, 16), 2),
         pl.BlockSpec((2, be, 8), lambda i: (0, i, 0))],
        [jax.ShapeDtypeStruct((e, 16), f32),
         jax.ShapeDtypeStruct((2, e, 8), f32)],
    )
    agg0 = _sc_scatter_add(m0, dst1, jnp.zeros((n, 8), f32), n)

    x1, tbl1, xd1, degc = _tc_call(
        _node1_body, (gn,),
        [(agg0, pl.BlockSpec((2, bn, 8), lambda i: (0, i, 0))),
         (xd0, _bspec((bn, 16), 2))],
        [lo0[:7], lo0[7:], PAD0, G1, SEL1],
        [_bspec((bn, 28), 2), _bspec((bn, 32), 2), _bspec((bn, 16), 2),
         _bspec((bn, 1), 2)],
        [jax.ShapeDtypeStruct((n, 28), f32), jax.ShapeDtypeStruct((n, 32), f32),
         jax.ShapeDtypeStruct((n, 16), f32), jax.ShapeDtypeStruct((n, 1), f32)],
    )

    gs1, gd1 = _sc_gather(tbl1, xd1, src1, dst1)
    m1, = _tc_call(
        _edge1_body, (ge,),
        [(eemb, _bspec((be, 16), 2)), (edge_sh, _bspec((be, 4), 2)),
         (gs1, _bspec((be, 32), 2)), (gd1, _bspec((be, 16), 2))],
        [f1W1[:16], f1W1[16:32], f1W1[32:], row(f1b1), f1W2, row(f1b2),
         f1W3, row(f1b3), R1, T1, Pa1, Pb1],
        [pl.BlockSpec((2, be, 12), lambda i: (0, i, 0))],
        [jax.ShapeDtypeStruct((2, e, 12), f32)],
    )
    agg1 = _sc_scatter_add(m1, dst1, jnp.zeros((n, 12), f32), n)

    out, = _tc_call(
        _node2_body, (gn,),
        [(agg1, pl.BlockSpec((2, bn, 12), lambda i: (0, i, 0))),
         (x1, _bspec((bn, 28), 2)), (degc, _bspec((bn, 1), 2)),
         (score_norm.reshape(n, 1), _bspec((bn, 1), 2))],
        [lo1[:10], lo1[10:], PAD1, Wt2],
        [_bspec((bn, 3), 2)],
        [jax.ShapeDtypeStruct((n, 3), f32)],
    )
    return out
